# baseline (speedup 1.0000x reference)
import functools

import jax
import jax.numpy as jnp
from jax import lax
from jax.experimental import pallas as pl
from jax.experimental.pallas import tpu as pltpu
from jax.experimental.pallas import tpu_sc as plsc

D_MODEL = 2048
OFFSET = 2
B, S = 4, 2048

NUM_CORES = 2
NUM_SUBCORES = 16
NW = NUM_CORES * NUM_SUBCORES
S_PER_W = S // NW
CHUNK = 8
NSG = S_PER_W // CHUNK
NSTEP = NSG * B
NBUF = 5
LEAD = 3
NPOS = 2
LANES = 16
COLS = D_MODEL // LANES

_mesh = plsc.VectorSubcoreMesh(core_axis_name="c", subcore_axis_name="s")


@functools.partial(
    pl.kernel,
    out_type=jax.ShapeDtypeStruct((B * S, D_MODEL), jnp.float32),
    mesh=_mesh,
    scratch_types=[
        pltpu.VMEM((B * S_PER_W,), jnp.int32),
        pltpu.VMEM((S_PER_W,), jnp.int32),
        [pltpu.VMEM((CHUNK, D_MODEL), jnp.float32) for _ in range(NBUF)],
        [pltpu.VMEM((CHUNK, D_MODEL), jnp.float32) for _ in range(NPOS)],
        [pltpu.SemaphoreType.DMA for _ in range(NBUF)],
        [pltpu.SemaphoreType.DMA for _ in range(NBUF)],
        [pltpu.SemaphoreType.DMA for _ in range(NPOS)],
    ],
)
def _embed_sc(ids_hbm, tok_hbm, pos_hbm, out_hbm,
              idx_v, pidx_v, bufs, posbs, gsems, osems, psems):
    wid = lax.axis_index("s") * NUM_CORES + lax.axis_index("c")
    s0 = wid * S_PER_W

    icps = [
        pltpu.async_copy(
            ids_hbm.at[pl.ds(b * S + s0, S_PER_W)],
            idx_v.at[pl.ds(b * S_PER_W, S_PER_W)],
            osems[b],
        )
        for b in range(B)
    ]
    lane = lax.iota(jnp.int32, LANES)
    for g in range(S_PER_W // LANES):
        pidx_v[pl.ds(g * LANES, LANES)] = lane + (s0 + OFFSET + g * LANES)
    for cp in icps:
        cp.wait()

    def gather(t, nb):
        sg, b = divmod(t, B)
        idx = idx_v.at[pl.ds(b * S_PER_W + sg * CHUNK, CHUNK)]
        return pltpu.async_copy(tok_hbm.at[idx], bufs[nb], gsems[nb])

    def gather_pos(sg):
        idx = pidx_v.at[pl.ds(sg * CHUNK, CHUNK)]
        return pltpu.async_copy(pos_hbm.at[idx], posbs[sg % NPOS], psems[sg % NPOS])

    def add_pos(nb, sg):
        buf = bufs[nb]
        posb = posbs[sg % NPOS]

        @plsc.parallel_loop(0, COLS, unroll=4)
        def _(c):
            cs = pl.ds(c * LANES, LANES)
            for r in range(CHUNK):
                plsc.addupdate(buf.at[r, cs], posb[r, cs])

    pcp = [gather_pos(sg) for sg in range(NPOS)]
    gcp = [None] * NBUF
    for u in range(LEAD + 1):
        gcp[u] = gather(u, u)
    ocp = [None] * NBUF
    for t in range(NSTEP):
        sg, b = divmod(t, B)
        nb = t % NBUF
        gcp[nb].wait()
        if b == 0:
            pcp[sg % NPOS].wait()
        add_pos(nb, sg)
        out_row = b * S + s0 + sg * CHUNK
        ocp[nb] = pltpu.async_copy(bufs[nb], out_hbm.at[pl.ds(out_row, CHUNK)], osems[nb])
        if b == B - 1 and sg + NPOS < NSG:
            pcp[sg % NPOS] = gather_pos(sg + NPOS)
        u = t + LEAD + 1
        if u < NSTEP:
            ub = u % NBUF
            if ocp[ub] is not None:
                ocp[ub].wait()
            gcp[ub] = gather(u, ub)
    for cp in ocp:
        if cp is not None:
            cp.wait()


def kernel(input_ids, embed_tokens, embed_positions):
    ids = input_ids.reshape(-1).astype(jnp.int32)
    out = _embed_sc(ids, embed_tokens, embed_positions)
    return out.reshape(B, S, D_MODEL)


# speedup vs baseline: 1.0726x; 1.0726x over previous
import functools

import jax
import jax.numpy as jnp
from jax import lax
from jax.experimental import pallas as pl
from jax.experimental.pallas import tpu as pltpu
from jax.experimental.pallas import tpu_sc as plsc

D_MODEL = 2048
OFFSET = 2
B, S = 4, 2048

NUM_CORES = 2
NUM_SUBCORES = 16
NW = NUM_CORES * NUM_SUBCORES
S_PER_W = S // NW
CHUNK = 8
NSG = S_PER_W // CHUNK
NSTEP = NSG * B
NBUF = 5
LEAD = 3
NPOS = 2
LANES = 16
COLS = D_MODEL // LANES

_mesh = plsc.VectorSubcoreMesh(core_axis_name="c", subcore_axis_name="s")


@functools.partial(
    pl.kernel,
    out_type=jax.ShapeDtypeStruct((B * S, D_MODEL), jnp.float32),
    mesh=_mesh,
    scratch_types=[
        pltpu.VMEM((B * S_PER_W,), jnp.int32),
        pltpu.VMEM((S_PER_W,), jnp.int32),
        [pltpu.VMEM((CHUNK, D_MODEL), jnp.float32) for _ in range(NBUF)],
        [pltpu.VMEM((CHUNK, D_MODEL), jnp.float32) for _ in range(NPOS)],
        [pltpu.SemaphoreType.DMA for _ in range(NBUF)],
        [pltpu.SemaphoreType.DMA for _ in range(NBUF)],
        [pltpu.SemaphoreType.DMA for _ in range(NPOS)],
    ],
)
def _embed_sc(ids_hbm, tok_hbm, pos_hbm, out_hbm,
              idx_v, pidx_v, bufs, posbs, gsems, osems, psems):
    wid = lax.axis_index("s") * NUM_CORES + lax.axis_index("c")
    s0 = wid * S_PER_W

    icps = [
        pltpu.async_copy(
            ids_hbm.at[pl.ds(b * S + s0, S_PER_W)],
            idx_v.at[pl.ds(b * S_PER_W, S_PER_W)],
            osems[b],
        )
        for b in range(B)
    ]
    lane = lax.iota(jnp.int32, LANES)
    for g in range(S_PER_W // LANES):
        pidx_v[pl.ds(g * LANES, LANES)] = lane + (s0 + OFFSET + g * LANES)
    for cp in icps:
        cp.wait()

    def gather(t, nb):
        sg, b = divmod(t, B)
        idx = idx_v.at[pl.ds(b * S_PER_W + sg * CHUNK, CHUNK)]
        return pltpu.async_copy(tok_hbm.at[idx], bufs[nb], gsems[nb])

    def gather_pos(sg):
        idx = pidx_v.at[pl.ds(sg * CHUNK, CHUNK)]
        return pltpu.async_copy(pos_hbm.at[idx], posbs[sg % NPOS], psems[sg % NPOS])

    def add_pos(nb, sg):
        buf = bufs[nb]
        posb = posbs[sg % NPOS]

        @plsc.parallel_loop(0, CHUNK * COLS, unroll=8)
        def _(i):
            r = lax.shift_right_logical(i, 7)
            cs = pl.ds((i & (COLS - 1)) * LANES, LANES)
            plsc.addupdate(buf.at[r, cs], posb[r, cs])

    pcp = [gather_pos(sg) for sg in range(NPOS)]
    gcp = [None] * NBUF
    for u in range(LEAD + 1):
        gcp[u] = gather(u, u)
    ocp = [None] * NBUF
    for t in range(NSTEP):
        sg, b = divmod(t, B)
        nb = t % NBUF
        gcp[nb].wait()
        if b == 0:
            pcp[sg % NPOS].wait()
        add_pos(nb, sg)
        out_row = b * S + s0 + sg * CHUNK
        ocp[nb] = pltpu.async_copy(bufs[nb], out_hbm.at[pl.ds(out_row, CHUNK)], osems[nb])
        if b == B - 1 and sg + NPOS < NSG:
            pcp[sg % NPOS] = gather_pos(sg + NPOS)
        u = t + LEAD + 1
        if u < NSTEP:
            ub = u % NBUF
            if ocp[ub] is not None:
                ocp[ub].wait()
            gcp[ub] = gather(u, ub)
    for cp in ocp:
        if cp is not None:
            cp.wait()


def kernel(input_ids, embed_tokens, embed_positions):
    ids = input_ids.reshape(-1).astype(jnp.int32)
    out = _embed_sc(ids, embed_tokens, embed_positions)
    return out.reshape(B, S, D_MODEL)
